# baseline (speedup 1.0000x reference)
import functools

import jax
import jax.numpy as jnp
import numpy as np
from jax import lax
from jax.experimental import pallas as pl
from jax.experimental.pallas import tpu as pltpu
from jax.experimental.pallas import tpu_sc as plsc

_NC = 2
_NS = 16
_NW = _NC * _NS
_K = 128


def _mesh():
    return plsc.VectorSubcoreMesh(
        core_axis_name="c", subcore_axis_name="s",
        num_cores=_NC, num_subcores=_NS)


def _pad_rows(n):
    rpt = -(-n // _NS)
    rpt = -(-rpt // 128) * 128
    return rpt, rpt * _NS


def _deg_call(dst, n):
    e = dst.shape[0]
    nchunks = e // _K
    assert nchunks * _K == e
    nfull, extra = divmod(nchunks, _NW)
    rpt, npad = _pad_rows(n)

    @functools.partial(
        pl.kernel,
        out_type=jax.ShapeDtypeStruct((_NC * npad,), jnp.float32),
        mesh=_mesh(),
        scratch_types=[
            pltpu.VMEM_SHARED((npad,), jnp.float32),
            [pltpu.VMEM((_K,), jnp.int32) for _ in range(2)],
            pltpu.VMEM((_K,), jnp.float32),
            pltpu.SemaphoreType.DMA,
        ],
    )
    def kern(dst_hbm, zvec_hbm, out_hbm, acc, didx, ones, sem):
        c = lax.axis_index("c")
        s = lax.axis_index("s")
        wid = s * _NC + c
        pltpu.sync_copy(zvec_hbm, acc.at[pl.ds(s * rpt, rpt)])
        for j in range(_K // 16):
            ones[pl.ds(j * 16, 16)] = jnp.ones((16,), jnp.float32)
        plsc.subcore_barrier()

        def step(i, b):
            @pl.when(i >= 2)
            def _():
                pltpu.make_async_copy(ones, acc.at[didx[b]], sem).wait()
            pltpu.sync_copy(dst_hbm.at[pl.ds((wid + i * _NW) * _K, _K)],
                            didx[b])
            pltpu.async_copy(ones, acc.at[didx[b]], sem, add=True)

        def body(j, carry):
            step(2 * j, 0)
            step(2 * j + 1, 1)
            return carry

        assert nfull % 2 == 0
        lax.fori_loop(0, nfull // 2, body, 0)
        for b in range(2):
            pltpu.make_async_copy(ones, acc.at[didx[b]], sem).wait()
        if extra:
            @pl.when(wid < extra)
            def _():
                pltpu.sync_copy(
                    dst_hbm.at[pl.ds((wid + nfull * _NW) * _K, _K)], didx[0])
                pltpu.sync_copy(ones, acc.at[didx[0]], add=True)
        plsc.subcore_barrier()
        pltpu.sync_copy(acc.at[pl.ds(s * rpt, rpt)],
                        out_hbm.at[pl.ds(c * npad + s * rpt, rpt)])

    zvec = jnp.zeros((rpt,), jnp.float32)
    return kern(dst, zvec).reshape(_NC, npad)


def _scatter_call(table, src, dst, n):
    e = src.shape[0]
    d = table.shape[1]
    assert d == 128
    nchunks = e // _K
    assert nchunks * _K == e
    nfull, extra = divmod(nchunks, _NW)
    rpt, npad = _pad_rows(n)

    @functools.partial(
        pl.kernel,
        out_type=jax.ShapeDtypeStruct((_NC, npad, d), jnp.float32),
        mesh=_mesh(),
        scratch_types=[
            pltpu.VMEM_SHARED((npad, d), jnp.float32),
            [pltpu.VMEM((_K,), jnp.int32) for _ in range(2)],
            [pltpu.VMEM((_K,), jnp.int32) for _ in range(2)],
            [pltpu.VMEM((_K, d), jnp.float32) for _ in range(2)],
            pltpu.SemaphoreType.DMA,
            pltpu.SemaphoreType.DMA,
        ],
    )
    def kern(tab_hbm, src_hbm, dst_hbm, zrows_hbm, out_hbm,
             acc, sidx, didx, rows, sem, ssem):
        c = lax.axis_index("c")
        s = lax.axis_index("s")
        wid = s * _NC + c

        def stage(i, b):
            g = (wid + i * _NW) * _K
            pltpu.sync_copy(src_hbm.at[pl.ds(g, _K)], sidx[b])
            pltpu.sync_copy(dst_hbm.at[pl.ds(g, _K)], didx[b])
            pltpu.async_copy(tab_hbm.at[sidx[b]], rows[b], sem)

        stage(0, 0)
        pltpu.sync_copy(zrows_hbm, acc.at[pl.ds(s * rpt, rpt)])
        plsc.subcore_barrier()

        def step(i, b, bn):
            @pl.when(i + 1 < nfull)
            def _():
                @pl.when(i >= 1)
                def _():
                    pltpu.make_async_copy(rows[bn], acc.at[didx[bn]],
                                          ssem).wait()
                stage(i + 1, bn)
            pltpu.make_async_copy(tab_hbm.at[sidx[b]], rows[b], sem).wait()
            pltpu.async_copy(rows[b], acc.at[didx[b]], ssem, add=True)

        def body(j, carry):
            step(2 * j, 0, 1)
            step(2 * j + 1, 1, 0)
            return carry

        assert nfull % 2 == 0
        lax.fori_loop(0, nfull // 2, body, 0)
        for b in range(2):
            pltpu.make_async_copy(rows[b], acc.at[didx[b]], ssem).wait()
        if extra:
            @pl.when(wid < extra)
            def _():
                stage(nfull, 0)
                pltpu.make_async_copy(tab_hbm.at[sidx[0]], rows[0],
                                      sem).wait()
                pltpu.sync_copy(rows[0], acc.at[didx[0]], add=True)
        plsc.subcore_barrier()
        pltpu.sync_copy(acc.at[pl.ds(s * rpt, rpt)],
                        out_hbm.at[c, pl.ds(s * rpt, rpt)])

    zrows = jnp.zeros((rpt, d), jnp.float32)
    return kern(table, src, dst, zrows)


_BN = 1000


def _dinv_of(degt_blk):
    deg = degt_blk[:, 0:1] + degt_blk[:, 1:2] + 1.0
    return lax.rsqrt(deg)


def _tc1a_body(x_ref, w_ref, t_ref):
    t_ref[...] = jnp.dot(x_ref[...], w_ref[...],
                         preferred_element_type=jnp.float32)


def _tc1b_body(t_ref, b_ref, degt_ref, g_ref, u_ref):
    dinv = _dinv_of(degt_ref[...])
    t = t_ref[...]
    g_ref[...] = dinv * t
    u_ref[...] = (dinv * dinv) * t + b_ref[...]


def _tc2_body(m_ref, u_ref, w_ref, b_ref, degt_ref, g_ref, u2_ref):
    dinv = _dinv_of(degt_ref[...])
    h1 = jnp.maximum(dinv * (m_ref[0] + m_ref[1]) + u_ref[...], 0.0)
    h2 = np.float32(0.9) * h1 + np.float32(0.1)
    t = jnp.dot(h2, w_ref[...], preferred_element_type=jnp.float32)
    dout = t.shape[1]
    gpad = jnp.concatenate(
        [dinv * t, jnp.zeros((t.shape[0], 128 - dout), jnp.float32)], axis=1)
    g_ref[...] = gpad
    u2_ref[...] = (dinv * dinv) * t + b_ref[...]


def _tc3_body(m_ref, u_ref, degt_ref, o_ref):
    dinv = _dinv_of(degt_ref[...])
    dout = u_ref.shape[1]
    msum = (m_ref[0] + m_ref[1])[:, :dout]
    pre = dinv * msum + u_ref[...]
    v = pre - jnp.max(pre, axis=1, keepdims=True)
    o_ref[...] = v - jnp.log(jnp.sum(jnp.exp(v), axis=1, keepdims=True))


def _tc1a(x, w1, n, din, dh):
    grid = (n // _BN,)
    return pl.pallas_call(
        _tc1a_body,
        grid=grid,
        in_specs=[
            pl.BlockSpec((_BN, din), lambda i: (i, 0)),
            pl.BlockSpec((din, dh), lambda i: (0, 0)),
        ],
        out_specs=pl.BlockSpec((_BN, dh), lambda i: (i, 0)),
        out_shape=jax.ShapeDtypeStruct((n, dh), jnp.float32),
    )(x, w1)


def _tc1b(t1, b1, degt, n, dh):
    grid = (n // _BN,)
    return pl.pallas_call(
        _tc1b_body,
        grid=grid,
        in_specs=[
            pl.BlockSpec((_BN, dh), lambda i: (i, 0)),
            pl.BlockSpec((1, dh), lambda i: (0, 0)),
            pl.BlockSpec((_BN, 2), lambda i: (i, 0)),
        ],
        out_specs=[
            pl.BlockSpec((_BN, dh), lambda i: (i, 0)),
            pl.BlockSpec((_BN, dh), lambda i: (i, 0)),
        ],
        out_shape=[
            jax.ShapeDtypeStruct((n, dh), jnp.float32),
            jax.ShapeDtypeStruct((n, dh), jnp.float32),
        ],
    )(t1, b1.reshape(1, dh), degt)


def _tc2(m1, u1, w2, b2, degt, n, dh, dout):
    grid = (n // _BN,)
    return pl.pallas_call(
        _tc2_body,
        grid=grid,
        in_specs=[
            pl.BlockSpec((_NC, _BN, dh), lambda i: (0, i, 0)),
            pl.BlockSpec((_BN, dh), lambda i: (i, 0)),
            pl.BlockSpec((dh, dout), lambda i: (0, 0)),
            pl.BlockSpec((1, dout), lambda i: (0, 0)),
            pl.BlockSpec((_BN, 2), lambda i: (i, 0)),
        ],
        out_specs=[
            pl.BlockSpec((_BN, 128), lambda i: (i, 0)),
            pl.BlockSpec((_BN, dout), lambda i: (i, 0)),
        ],
        out_shape=[
            jax.ShapeDtypeStruct((n, 128), jnp.float32),
            jax.ShapeDtypeStruct((n, dout), jnp.float32),
        ],
    )(m1, u1, w2, b2.reshape(1, dout), degt)


def _tc3(m2, u2, degt, n, dout):
    grid = (n // _BN,)
    return pl.pallas_call(
        _tc3_body,
        grid=grid,
        in_specs=[
            pl.BlockSpec((_NC, _BN, 128), lambda i: (0, i, 0)),
            pl.BlockSpec((_BN, dout), lambda i: (i, 0)),
            pl.BlockSpec((_BN, 2), lambda i: (i, 0)),
        ],
        out_specs=pl.BlockSpec((_BN, dout), lambda i: (i, 0)),
        out_shape=jax.ShapeDtypeStruct((n, dout), jnp.float32),
    )(m2, u2, degt)


def kernel(x, edge_index, edge_weight, W1, b1, W2, b2):
    n, din = x.shape
    dh = W1.shape[1]
    dout = W2.shape[1]
    src_i = edge_index[0]
    dst_i = edge_index[1]

    t1 = _tc1a(x, W1, n, din, dh)
    deg_parts = _deg_call(dst_i, n)
    degt = jnp.transpose(deg_parts)

    g1, u1 = _tc1b(t1, b1, degt, n, dh)
    m1 = _scatter_call(g1, src_i, dst_i, n)
    g2, u2 = _tc2(m1, u1, W2, b2, degt, n, dh, dout)
    m2 = _scatter_call(g2, src_i, dst_i, n)
    return _tc3(m2, u2, degt, n, dout)


# speedup vs baseline: 1.0054x; 1.0054x over previous
import functools

import jax
import jax.numpy as jnp
import numpy as np
from jax import lax
from jax.experimental import pallas as pl
from jax.experimental.pallas import tpu as pltpu
from jax.experimental.pallas import tpu_sc as plsc

_NC = 2
_NS = 16
_NW = _NC * _NS
_K = 128


def _mesh():
    return plsc.VectorSubcoreMesh(
        core_axis_name="c", subcore_axis_name="s",
        num_cores=_NC, num_subcores=_NS)


def _pad_rows(n):
    rpt = -(-n // _NS)
    rpt = -(-rpt // 128) * 128
    return rpt, rpt * _NS


def _deg_call(dst, n):
    e = dst.shape[0]
    nchunks = e // _K
    assert nchunks * _K == e
    nfull, extra = divmod(nchunks, _NW)
    rpt, npad = _pad_rows(n)

    @functools.partial(
        pl.kernel,
        out_type=jax.ShapeDtypeStruct((_NC * npad,), jnp.float32),
        mesh=_mesh(),
        scratch_types=[
            pltpu.VMEM_SHARED((npad,), jnp.float32),
            [pltpu.VMEM((_K,), jnp.int32) for _ in range(2)],
            pltpu.VMEM((_K,), jnp.float32),
            pltpu.SemaphoreType.DMA,
        ],
    )
    def kern(dst_hbm, zvec_hbm, out_hbm, acc, didx, ones, sem):
        c = lax.axis_index("c")
        s = lax.axis_index("s")
        wid = s * _NC + c
        pltpu.sync_copy(zvec_hbm, acc.at[pl.ds(s * rpt, rpt)])
        for j in range(_K // 16):
            ones[pl.ds(j * 16, 16)] = jnp.ones((16,), jnp.float32)
        plsc.subcore_barrier()

        def step(i, b):
            @pl.when(i >= 2)
            def _():
                pltpu.make_async_copy(ones, acc.at[didx[b]], sem).wait()
            pltpu.sync_copy(dst_hbm.at[pl.ds((wid + i * _NW) * _K, _K)],
                            didx[b])
            pltpu.async_copy(ones, acc.at[didx[b]], sem, add=True)

        def body(j, carry):
            step(2 * j, 0)
            step(2 * j + 1, 1)
            return carry

        assert nfull % 2 == 0
        lax.fori_loop(0, nfull // 2, body, 0)
        for b in range(2):
            pltpu.make_async_copy(ones, acc.at[didx[b]], sem).wait()
        if extra:
            @pl.when(wid < extra)
            def _():
                pltpu.sync_copy(
                    dst_hbm.at[pl.ds((wid + nfull * _NW) * _K, _K)], didx[0])
                pltpu.sync_copy(ones, acc.at[didx[0]], add=True)
        plsc.subcore_barrier()
        pltpu.sync_copy(acc.at[pl.ds(s * rpt, rpt)],
                        out_hbm.at[pl.ds(c * npad + s * rpt, rpt)])

    zvec = jnp.zeros((rpt,), jnp.float32)
    return kern(dst, zvec).reshape(_NC, npad)


def _scatter_call(table, src, dst, n):
    e = src.shape[0]
    d = table.shape[1]
    assert d == 128
    nchunks = e // _K
    assert nchunks * _K == e
    nfull, extra = divmod(nchunks, _NW)
    assert nfull % 3 == 0
    rpt = -(-(-(-n // _NS)) // 8) * 8
    npad = rpt * _NS

    @functools.partial(
        pl.kernel,
        out_type=jax.ShapeDtypeStruct((_NC, npad, d), jnp.float32),
        mesh=_mesh(),
        scratch_types=[
            pltpu.VMEM_SHARED((npad, d), jnp.float32),
            [pltpu.VMEM((_K,), jnp.int32) for _ in range(3)],
            [pltpu.VMEM((_K,), jnp.int32) for _ in range(3)],
            [pltpu.VMEM((_K, d), jnp.float32) for _ in range(3)],
            pltpu.SemaphoreType.DMA,
            pltpu.SemaphoreType.DMA,
        ],
    )
    def kern(tab_hbm, src_hbm, dst_hbm, zrows_hbm, out_hbm,
             acc, sidx, didx, rows, sem, ssem):
        c = lax.axis_index("c")
        s = lax.axis_index("s")
        wid = s * _NC + c

        def stage(i, b):
            g = (wid + i * _NW) * _K
            pltpu.sync_copy(src_hbm.at[pl.ds(g, _K)], sidx[b])
            pltpu.sync_copy(dst_hbm.at[pl.ds(g, _K)], didx[b])
            pltpu.async_copy(tab_hbm.at[sidx[b]], rows[b], sem)

        stage(0, 0)
        stage(1, 1)
        pltpu.sync_copy(zrows_hbm, acc.at[pl.ds(s * rpt, rpt)])
        plsc.subcore_barrier()

        def step(i, b, b2):
            @pl.when(i + 2 < nfull)
            def _():
                @pl.when(i >= 1)
                def _():
                    pltpu.make_async_copy(rows[b2], acc.at[didx[b2]],
                                          ssem).wait()
                stage(i + 2, b2)
            pltpu.make_async_copy(tab_hbm.at[sidx[b]], rows[b], sem).wait()
            pltpu.async_copy(rows[b], acc.at[didx[b]], ssem, add=True)

        def body(j, carry):
            step(3 * j, 0, 2)
            step(3 * j + 1, 1, 0)
            step(3 * j + 2, 2, 1)
            return carry

        lax.fori_loop(0, nfull // 3, body, 0)
        for b in range(3):
            pltpu.make_async_copy(rows[b], acc.at[didx[b]], ssem).wait()
        if extra:
            @pl.when(wid < extra)
            def _():
                stage(nfull, 0)
                pltpu.make_async_copy(tab_hbm.at[sidx[0]], rows[0],
                                      sem).wait()
                pltpu.sync_copy(rows[0], acc.at[didx[0]], add=True)
        plsc.subcore_barrier()
        pltpu.sync_copy(acc.at[pl.ds(s * rpt, rpt)],
                        out_hbm.at[c, pl.ds(s * rpt, rpt)])

    zrows = jnp.zeros((rpt, d), jnp.float32)
    return kern(table, src, dst, zrows)


_BN = 1000


def _dinv_of(degt_blk):
    deg = degt_blk[:, 0:1] + degt_blk[:, 1:2] + 1.0
    return lax.rsqrt(deg)


def _tc1a_body(x_ref, w_ref, t_ref):
    t_ref[...] = jnp.dot(x_ref[...], w_ref[...],
                         preferred_element_type=jnp.float32)


def _tc1b_body(t_ref, b_ref, degt_ref, g_ref, u_ref):
    dinv = _dinv_of(degt_ref[...])
    t = t_ref[...]
    g_ref[...] = dinv * t
    u_ref[...] = (dinv * dinv) * t + b_ref[...]


def _tc2_body(m_ref, u_ref, w_ref, b_ref, degt_ref, g_ref, u2_ref):
    dinv = _dinv_of(degt_ref[...])
    h1 = jnp.maximum(dinv * (m_ref[0] + m_ref[1]) + u_ref[...], 0.0)
    h2 = np.float32(0.9) * h1 + np.float32(0.1)
    t = jnp.dot(h2, w_ref[...], preferred_element_type=jnp.float32)
    dout = t.shape[1]
    gpad = jnp.concatenate(
        [dinv * t, jnp.zeros((t.shape[0], 128 - dout), jnp.float32)], axis=1)
    g_ref[...] = gpad
    u2_ref[...] = (dinv * dinv) * t + b_ref[...]


def _tc3_body(m_ref, u_ref, degt_ref, o_ref):
    dinv = _dinv_of(degt_ref[...])
    dout = u_ref.shape[1]
    msum = (m_ref[0] + m_ref[1])[:, :dout]
    pre = dinv * msum + u_ref[...]
    v = pre - jnp.max(pre, axis=1, keepdims=True)
    o_ref[...] = v - jnp.log(jnp.sum(jnp.exp(v), axis=1, keepdims=True))


def _tc1a(x, w1, n, din, dh):
    grid = (n // _BN,)
    return pl.pallas_call(
        _tc1a_body,
        grid=grid,
        in_specs=[
            pl.BlockSpec((_BN, din), lambda i: (i, 0)),
            pl.BlockSpec((din, dh), lambda i: (0, 0)),
        ],
        out_specs=pl.BlockSpec((_BN, dh), lambda i: (i, 0)),
        out_shape=jax.ShapeDtypeStruct((n, dh), jnp.float32),
    )(x, w1)


def _tc1b(t1, b1, degt, n, dh):
    grid = (n // _BN,)
    return pl.pallas_call(
        _tc1b_body,
        grid=grid,
        in_specs=[
            pl.BlockSpec((_BN, dh), lambda i: (i, 0)),
            pl.BlockSpec((1, dh), lambda i: (0, 0)),
            pl.BlockSpec((_BN, 2), lambda i: (i, 0)),
        ],
        out_specs=[
            pl.BlockSpec((_BN, dh), lambda i: (i, 0)),
            pl.BlockSpec((_BN, dh), lambda i: (i, 0)),
        ],
        out_shape=[
            jax.ShapeDtypeStruct((n, dh), jnp.float32),
            jax.ShapeDtypeStruct((n, dh), jnp.float32),
        ],
    )(t1, b1.reshape(1, dh), degt)


def _tc2(m1, u1, w2, b2, degt, n, dh, dout):
    grid = (n // _BN,)
    return pl.pallas_call(
        _tc2_body,
        grid=grid,
        in_specs=[
            pl.BlockSpec((_NC, _BN, dh), lambda i: (0, i, 0)),
            pl.BlockSpec((_BN, dh), lambda i: (i, 0)),
            pl.BlockSpec((dh, dout), lambda i: (0, 0)),
            pl.BlockSpec((1, dout), lambda i: (0, 0)),
            pl.BlockSpec((_BN, 2), lambda i: (i, 0)),
        ],
        out_specs=[
            pl.BlockSpec((_BN, 128), lambda i: (i, 0)),
            pl.BlockSpec((_BN, dout), lambda i: (i, 0)),
        ],
        out_shape=[
            jax.ShapeDtypeStruct((n, 128), jnp.float32),
            jax.ShapeDtypeStruct((n, dout), jnp.float32),
        ],
    )(m1, u1, w2, b2.reshape(1, dout), degt)


def _tc3(m2, u2, degt, n, dout):
    grid = (n // _BN,)
    return pl.pallas_call(
        _tc3_body,
        grid=grid,
        in_specs=[
            pl.BlockSpec((_NC, _BN, 128), lambda i: (0, i, 0)),
            pl.BlockSpec((_BN, dout), lambda i: (i, 0)),
            pl.BlockSpec((_BN, 2), lambda i: (i, 0)),
        ],
        out_specs=pl.BlockSpec((_BN, dout), lambda i: (i, 0)),
        out_shape=jax.ShapeDtypeStruct((n, dout), jnp.float32),
    )(m2, u2, degt)


def kernel(x, edge_index, edge_weight, W1, b1, W2, b2):
    n, din = x.shape
    dh = W1.shape[1]
    dout = W2.shape[1]
    src_i = edge_index[0]
    dst_i = edge_index[1]

    t1 = _tc1a(x, W1, n, din, dh)
    deg_parts = _deg_call(dst_i, n)
    degt = jnp.transpose(deg_parts)

    g1, u1 = _tc1b(t1, b1, degt, n, dh)
    m1 = _scatter_call(g1, src_i, dst_i, n)
    g2, u2 = _tc2(m1, u1, W2, b2, degt, n, dh, dout)
    m2 = _scatter_call(g2, src_i, dst_i, n)
    return _tc3(m2, u2, degt, n, dout)


# speedup vs baseline: 1.0075x; 1.0022x over previous
import functools

import jax
import jax.numpy as jnp
import numpy as np
from jax import lax
from jax.experimental import pallas as pl
from jax.experimental.pallas import tpu as pltpu
from jax.experimental.pallas import tpu_sc as plsc

_NC = 2
_NS = 16
_NW = _NC * _NS
_K = 128


def _mesh():
    return plsc.VectorSubcoreMesh(
        core_axis_name="c", subcore_axis_name="s",
        num_cores=_NC, num_subcores=_NS)


def _pad_rows(n):
    rpt = -(-n // _NS)
    rpt = -(-rpt // 128) * 128
    return rpt, rpt * _NS


def _deg_call(dst, n):
    e = dst.shape[0]
    nchunks = e // _K
    assert nchunks * _K == e
    nfull, extra = divmod(nchunks, _NW)
    rpt, npad = _pad_rows(n)

    @functools.partial(
        pl.kernel,
        out_type=jax.ShapeDtypeStruct((_NC * npad,), jnp.float32),
        mesh=_mesh(),
        scratch_types=[
            pltpu.VMEM_SHARED((npad,), jnp.float32),
            [pltpu.VMEM((_K,), jnp.int32) for _ in range(2)],
            pltpu.VMEM((_K,), jnp.float32),
            pltpu.SemaphoreType.DMA,
        ],
    )
    def kern(dst_hbm, zvec_hbm, out_hbm, acc, didx, ones, sem):
        c = lax.axis_index("c")
        s = lax.axis_index("s")
        wid = s * _NC + c
        pltpu.sync_copy(zvec_hbm, acc.at[pl.ds(s * rpt, rpt)])
        for j in range(_K // 16):
            ones[pl.ds(j * 16, 16)] = jnp.ones((16,), jnp.float32)
        plsc.subcore_barrier()

        def step(i, b):
            @pl.when(i >= 2)
            def _():
                pltpu.make_async_copy(ones, acc.at[didx[b]], sem).wait()
            pltpu.sync_copy(dst_hbm.at[pl.ds((wid + i * _NW) * _K, _K)],
                            didx[b])
            pltpu.async_copy(ones, acc.at[didx[b]], sem, add=True)

        def body(j, carry):
            step(2 * j, 0)
            step(2 * j + 1, 1)
            return carry

        assert nfull % 2 == 0
        lax.fori_loop(0, nfull // 2, body, 0)
        for b in range(2):
            pltpu.make_async_copy(ones, acc.at[didx[b]], sem).wait()
        if extra:
            @pl.when(wid < extra)
            def _():
                pltpu.sync_copy(
                    dst_hbm.at[pl.ds((wid + nfull * _NW) * _K, _K)], didx[0])
                pltpu.sync_copy(ones, acc.at[didx[0]], add=True)
        plsc.subcore_barrier()
        pltpu.sync_copy(acc.at[pl.ds(s * rpt, rpt)],
                        out_hbm.at[pl.ds(c * npad + s * rpt, rpt)])

    zvec = jnp.zeros((rpt,), jnp.float32)
    return kern(dst, zvec).reshape(_NC, npad)


def _scatter_call(table, src, dst, n):
    e = src.shape[0]
    d = table.shape[1]
    assert d == 128
    nchunks = e // _K
    assert nchunks * _K == e
    nfull, extra = divmod(nchunks, _NW)
    assert nfull % 3 == 0
    rpt = -(-(-(-n // _NS)) // 8) * 8
    npad = rpt * _NS

    @functools.partial(
        pl.kernel,
        out_type=jax.ShapeDtypeStruct((_NC, npad, d), jnp.float32),
        mesh=_mesh(),
        scratch_types=[
            pltpu.VMEM_SHARED((npad, d), jnp.float32),
            [pltpu.VMEM((_K,), jnp.int32) for _ in range(3)],
            [pltpu.VMEM((_K,), jnp.int32) for _ in range(3)],
            [pltpu.VMEM((_K, d), jnp.float32) for _ in range(3)],
            pltpu.SemaphoreType.DMA,
            pltpu.SemaphoreType.DMA,
        ],
    )
    def kern(tab_hbm, src_hbm, dst_hbm, zrows_hbm, out_hbm,
             acc, sidx, didx, rows, sem, ssem):
        c = lax.axis_index("c")
        s = lax.axis_index("s")
        wid = s * _NC + c

        def stage(i, b):
            g = (wid + i * _NW) * _K
            pltpu.sync_copy(src_hbm.at[pl.ds(g, _K)], sidx[b])
            pltpu.sync_copy(dst_hbm.at[pl.ds(g, _K)], didx[b])
            pltpu.async_copy(tab_hbm.at[sidx[b]], rows[b], sem)

        stage(0, 0)
        stage(1, 1)
        pltpu.sync_copy(zrows_hbm, acc.at[pl.ds(s * rpt, rpt)])
        plsc.subcore_barrier()

        def step(i, b, b2):
            @pl.when(i + 2 < nfull)
            def _():
                @pl.when(i >= 1)
                def _():
                    pltpu.make_async_copy(rows[b2], acc.at[didx[b2]],
                                          ssem).wait()
                stage(i + 2, b2)
            pltpu.make_async_copy(tab_hbm.at[sidx[b]], rows[b], sem).wait()
            pltpu.async_copy(rows[b], acc.at[didx[b]], ssem, add=True)

        def body(j, carry):
            step(3 * j, 0, 2)
            step(3 * j + 1, 1, 0)
            step(3 * j + 2, 2, 1)
            return carry

        lax.fori_loop(0, nfull // 3, body, 0)
        for b in range(3):
            pltpu.make_async_copy(rows[b], acc.at[didx[b]], ssem).wait()
        if extra:
            @pl.when(wid < extra)
            def _():
                stage(nfull, 0)
                pltpu.make_async_copy(tab_hbm.at[sidx[0]], rows[0],
                                      sem).wait()
                pltpu.sync_copy(rows[0], acc.at[didx[0]], add=True)
        plsc.subcore_barrier()
        pltpu.sync_copy(acc.at[pl.ds(s * rpt, rpt)],
                        out_hbm.at[c, pl.ds(s * rpt, rpt)])

    zrows = jnp.zeros((rpt, d), jnp.float32)
    return kern(table, src, dst, zrows)


_BN = 1000


def _dinv_of(degt_blk):
    deg = degt_blk[:, 0:1] + degt_blk[:, 1:2] + 1.0
    return lax.rsqrt(deg)


def _tc1_body(x_ref, w_ref, b_ref, degt_ref, g_ref, u_ref):
    dinv = _dinv_of(degt_ref[...])
    t = jnp.dot(x_ref[...], w_ref[...], preferred_element_type=jnp.float32)
    g_ref[...] = dinv * t
    u_ref[...] = (dinv * dinv) * t + b_ref[...]


def _tc2_body(m_ref, u_ref, w_ref, b_ref, degt_ref, g_ref, u2_ref):
    dinv = _dinv_of(degt_ref[...])
    h1 = jnp.maximum(dinv * (m_ref[0] + m_ref[1]) + u_ref[...], 0.0)
    h2 = np.float32(0.9) * h1 + np.float32(0.1)
    t = jnp.dot(h2, w_ref[...], preferred_element_type=jnp.float32)
    dout = t.shape[1]
    gpad = jnp.concatenate(
        [dinv * t, jnp.zeros((t.shape[0], 128 - dout), jnp.float32)], axis=1)
    g_ref[...] = gpad
    u2_ref[...] = (dinv * dinv) * t + b_ref[...]


def _tc3_body(m_ref, u_ref, degt_ref, o_ref):
    dinv = _dinv_of(degt_ref[...])
    dout = u_ref.shape[1]
    msum = (m_ref[0] + m_ref[1])[:, :dout]
    pre = dinv * msum + u_ref[...]
    v = pre - jnp.max(pre, axis=1, keepdims=True)
    o_ref[...] = v - jnp.log(jnp.sum(jnp.exp(v), axis=1, keepdims=True))


def _tc1(x, w1, b1, degt, n, din, dh):
    grid = (n // _BN,)
    return pl.pallas_call(
        _tc1_body,
        grid=grid,
        in_specs=[
            pl.BlockSpec((_BN, din), lambda i: (i, 0)),
            pl.BlockSpec((din, dh), lambda i: (0, 0)),
            pl.BlockSpec((1, dh), lambda i: (0, 0)),
            pl.BlockSpec((_BN, 2), lambda i: (i, 0)),
        ],
        out_specs=[
            pl.BlockSpec((_BN, dh), lambda i: (i, 0)),
            pl.BlockSpec((_BN, dh), lambda i: (i, 0)),
        ],
        out_shape=[
            jax.ShapeDtypeStruct((n, dh), jnp.float32),
            jax.ShapeDtypeStruct((n, dh), jnp.float32),
        ],
    )(x, w1, b1.reshape(1, dh), degt)


def _tc2(m1, u1, w2, b2, degt, n, dh, dout):
    grid = (n // _BN,)
    return pl.pallas_call(
        _tc2_body,
        grid=grid,
        in_specs=[
            pl.BlockSpec((_NC, _BN, dh), lambda i: (0, i, 0)),
            pl.BlockSpec((_BN, dh), lambda i: (i, 0)),
            pl.BlockSpec((dh, dout), lambda i: (0, 0)),
            pl.BlockSpec((1, dout), lambda i: (0, 0)),
            pl.BlockSpec((_BN, 2), lambda i: (i, 0)),
        ],
        out_specs=[
            pl.BlockSpec((_BN, 128), lambda i: (i, 0)),
            pl.BlockSpec((_BN, dout), lambda i: (i, 0)),
        ],
        out_shape=[
            jax.ShapeDtypeStruct((n, 128), jnp.float32),
            jax.ShapeDtypeStruct((n, dout), jnp.float32),
        ],
    )(m1, u1, w2, b2.reshape(1, dout), degt)


def _tc3(m2, u2, degt, n, dout):
    grid = (n // _BN,)
    return pl.pallas_call(
        _tc3_body,
        grid=grid,
        in_specs=[
            pl.BlockSpec((_NC, _BN, 128), lambda i: (0, i, 0)),
            pl.BlockSpec((_BN, dout), lambda i: (i, 0)),
            pl.BlockSpec((_BN, 2), lambda i: (i, 0)),
        ],
        out_specs=pl.BlockSpec((_BN, dout), lambda i: (i, 0)),
        out_shape=jax.ShapeDtypeStruct((n, dout), jnp.float32),
    )(m2, u2, degt)


def kernel(x, edge_index, edge_weight, W1, b1, W2, b2):
    n, din = x.shape
    dh = W1.shape[1]
    dout = W2.shape[1]
    src_i = edge_index[0]
    dst_i = edge_index[1]

    deg_parts = _deg_call(dst_i, n)
    degt = jnp.transpose(deg_parts)

    g1, u1 = _tc1(x, W1, b1, degt, n, din, dh)
    m1 = _scatter_call(g1, src_i, dst_i, n)
    g2, u2 = _tc2(m1, u1, W2, b2, degt, n, dh, dout)
    m2 = _scatter_call(g2, src_i, dst_i, n)
    return _tc3(m2, u2, degt, n, dout)
